# baseline (speedup 1.0000x reference)
import functools

import jax
import jax.numpy as jnp
from jax import lax
from jax.experimental import pallas as pl
from jax.experimental.pallas import tpu as pltpu
from jax.experimental.pallas import tpu_sc as plsc

_N = 16 * 640 * 640
_COLS = 640
_ROWS = _N // _COLS
_BLK = 256
_NBINS = 2048
_HI = 100.001
_TILES = 32
_TROWS = _ROWS // _TILES
_CROWS = 16
_CHUNK = _CROWS * _COLS
_NCHUNKS = _TROWS // _CROWS
_RUNITS = _COLS // 16
_UNROLL = 8
_HWORDS = _NBINS * 16


def _loss_body(pred_ref, map_ref, mask_ref, w_ref, neg_ref, stats_ref):
    p = pred_ref[...]
    m = map_ref[...]
    valid = mask_ref[...]
    w = w_ref[...]
    log_p = jnp.maximum(jnp.log(p), -100.0)
    log_1p = jnp.maximum(jnp.log(1.0 - p), -100.0)
    loss = -w * (m * log_p + (1.0 - m) * log_1p)
    pos_area = m * valid
    neg_area = (1.0 - m) * valid
    neg_ref[...] = loss * neg_area
    lane = lax.broadcasted_iota(jnp.int32, (1, 128), 1)
    row = jnp.where(lane == 0, jnp.sum(loss * pos_area), 0.0)
    row += jnp.where(lane == 1, jnp.sum((pos_area > 0.5).astype(jnp.float32)), 0.0)
    row += jnp.where(lane == 2, jnp.sum((neg_area > 0.5).astype(jnp.float32)), 0.0)

    @pl.when(pl.program_id(0) == 0)
    def _():
        stats_ref[...] = jnp.zeros_like(stats_ref)

    stats_ref[...] += row


def _loss_call(pred2, map2, mask2, w2):
    return pl.pallas_call(
        _loss_body,
        grid=(_ROWS // _BLK,),
        in_specs=[pl.BlockSpec((_BLK, _COLS), lambda i: (i, 0))] * 4,
        out_specs=[
            pl.BlockSpec((_BLK, _COLS), lambda i: (i, 0)),
            pl.BlockSpec((1, 128), lambda i: (0, 0)),
        ],
        out_shape=[
            jax.ShapeDtypeStruct((_ROWS, _COLS), jnp.float32),
            jax.ShapeDtypeStruct((1, 128), jnp.float32),
        ],
    )(pred2, map2, mask2, w2)


def _hist_body(neg_hbm, params_hbm, out_hbm, stage, buf0, buf1, pv,
               sem0, sem1):
    wid = lax.axis_index("c") * 16 + lax.axis_index("s")
    pltpu.sync_copy(params_hbm, pv)
    inv = pv[0]

    @plsc.parallel_loop(0, 2 * _HWORDS // 16, 1, unroll=_UNROLL)
    def _(i):
        stage[pl.ds(i * 16, 16)] = jnp.zeros((16,), jnp.float32)

    lanes = lax.iota(jnp.int32, 16)
    ones = jnp.ones((16,), jnp.float32)
    bufs = (buf0, buf1)
    sems = (sem0, sem1)

    def _start(g, b):
        row0 = pl.multiple_of(wid * _TROWS + g * _CROWS, _CROWS)
        pltpu.async_copy(neg_hbm.at[pl.ds(row0, _CROWS)], bufs[b], sems[b])

    def _wait(b):
        pltpu.make_async_copy(neg_hbm.at[pl.ds(0, _CROWS)], bufs[b],
                              sems[b]).wait()

    _start(0, 0)
    _start(1, 1)

    def _process(buf):
        @plsc.parallel_loop(0, _CROWS, 1, unroll=2)
        def _(r):
            for u in range(_RUNITS):
                v = buf[r, pl.ds(u * 16, 16)]
                b = jnp.maximum(jnp.minimum(v * inv, float(_NBINS - 1)), 0.0)
                idx = b.astype(jnp.int32) * 16 + lanes
                plsc.addupdate_scatter(stage, [idx], ones)
                plsc.addupdate_scatter(stage, [idx + _HWORDS], v)

    def cbody(g, _):
        for b in range(2):
            _wait(b)
            _process(bufs[b])

            @pl.when(g + b + 2 < _NCHUNKS)
            def _():
                _start(g + b + 2, b)

        return 0

    lax.fori_loop(0, _NCHUNKS // 2, lambda g, c: cbody(2 * g, c), 0)
    if _NCHUNKS % 2:
        _wait(0)
        _process(bufs[0])
    pltpu.sync_copy(stage, out_hbm.at[wid])


_hist_call = functools.partial(
    pl.kernel,
    out_type=jax.ShapeDtypeStruct((_TILES, 2 * _HWORDS), jnp.float32),
    mesh=plsc.VectorSubcoreMesh(core_axis_name="c", subcore_axis_name="s",
                                num_cores=2),
    compiler_params=pltpu.CompilerParams(needs_layout_passes=False),
    scratch_types=[
        pltpu.VMEM((2 * _HWORDS,), jnp.float32),
        pltpu.VMEM((_CROWS, _COLS), jnp.float32),
        pltpu.VMEM((_CROWS, _COLS), jnp.float32),
        pltpu.VMEM((8, 16), jnp.float32),
        pltpu.SemaphoreType.DMA,
        pltpu.SemaphoreType.DMA,
    ],
)(_hist_body)


def _params(*vals):
    cols = [jnp.broadcast_to(jnp.asarray(v, jnp.float32), (16,)) for v in vals]
    cols += [jnp.zeros(16, jnp.float32)] * (8 - len(cols))
    return jnp.stack(cols)


def _suffix(x):
    return jnp.concatenate([jnp.cumsum(x[::-1])[::-1], jnp.zeros(1, x.dtype)])


def kernel(prob_pred, prob_map, prob_mask, prob_weight):
    pred2 = prob_pred.reshape(_ROWS, _COLS)
    map2 = prob_map.reshape(_ROWS, _COLS)
    mask2 = prob_mask.reshape(_ROWS, _COLS)
    w2 = prob_weight.reshape(_ROWS, _COLS)

    neg2d, stats = _loss_call(pred2, map2, mask2, w2)
    pos_sum = stats[0, 0]
    pc = stats[0, 1].astype(jnp.int32)
    nc = stats[0, 2].astype(jnp.int32)
    k = jnp.minimum(nc, pc * 3)
    kf = k.astype(jnp.float32)

    inv_w = jnp.float32(_NBINS / _HI)
    out = _hist_call(neg2d, _params(inv_w))
    h = out.reshape(_TILES, 2, _NBINS, 16)
    cnt = h[:, 0].sum(axis=(0, 2))
    sm = h[:, 1].sum(axis=(0, 2))
    s = _suffix(cnt)
    ss = _suffix(sm)
    c = jnp.sum(s[:_NBINS] >= kf).astype(jnp.int32) - 1
    rem = kf - s[c + 1]
    avg = sm[c] / jnp.maximum(cnt[c], 1.0)
    topk = ss[c + 1] + rem * avg

    denom = (pc + k).astype(jnp.float32) + jnp.float32(1e-6)
    return (pos_sum + topk) / denom


# speedup vs baseline: 1.0791x; 1.0791x over previous
import functools

import jax
import jax.numpy as jnp
from jax import lax
from jax.experimental import pallas as pl
from jax.experimental.pallas import tpu as pltpu
from jax.experimental.pallas import tpu_sc as plsc

_N = 16 * 640 * 640
_COLS = 640
_ROWS = _N // _COLS
_BLK = 1024
_NBINS = 2048
_HI = 100.001
_TILES = 32
_TROWS = _ROWS // _TILES
_CROWS = 16
_CHUNK = _CROWS * _COLS
_NCHUNKS = _TROWS // _CROWS
_RUNITS = _COLS // 16
_UNROLL = 8
_HWORDS = _NBINS * 16


def _loss_body(pred_ref, map_ref, mask_ref, w_ref, neg_ref, stats_ref):
    p = pred_ref[...]
    m = map_ref[...]
    valid = mask_ref[...]
    w = w_ref[...]
    log_p = jnp.maximum(jnp.log(p), -100.0)
    log_1p = jnp.maximum(jnp.log(1.0 - p), -100.0)
    loss = -w * (m * log_p + (1.0 - m) * log_1p)
    pos_area = m * valid
    neg_area = (1.0 - m) * valid
    neg_ref[...] = loss * neg_area
    lane = lax.broadcasted_iota(jnp.int32, (1, 128), 1)
    row = jnp.where(lane == 0, jnp.sum(loss * pos_area), 0.0)
    row += jnp.where(lane == 1, jnp.sum((pos_area > 0.5).astype(jnp.float32)), 0.0)
    row += jnp.where(lane == 2, jnp.sum((neg_area > 0.5).astype(jnp.float32)), 0.0)

    @pl.when(pl.program_id(0) == 0)
    def _():
        stats_ref[...] = jnp.zeros_like(stats_ref)

    stats_ref[...] += row


def _loss_call(pred2, map2, mask2, w2):
    return pl.pallas_call(
        _loss_body,
        grid=(_ROWS // _BLK,),
        in_specs=[pl.BlockSpec((_BLK, _COLS), lambda i: (i, 0))] * 4,
        out_specs=[
            pl.BlockSpec((_BLK, _COLS), lambda i: (i, 0)),
            pl.BlockSpec((1, 128), lambda i: (0, 0)),
        ],
        out_shape=[
            jax.ShapeDtypeStruct((_ROWS, _COLS), jnp.float32),
            jax.ShapeDtypeStruct((1, 128), jnp.float32),
        ],
    )(pred2, map2, mask2, w2)


def _hist_body(neg_hbm, params_hbm, out_hbm, stage, buf0, buf1, pv,
               sem0, sem1):
    wid = lax.axis_index("c") * 16 + lax.axis_index("s")
    pltpu.sync_copy(params_hbm, pv)
    inv = pv[0]

    @plsc.parallel_loop(0, 2 * _HWORDS // 16, 1, unroll=_UNROLL)
    def _(i):
        stage[pl.ds(i * 16, 16)] = jnp.zeros((16,), jnp.float32)

    lanes = lax.iota(jnp.int32, 16)
    ones = jnp.ones((16,), jnp.float32)
    bufs = (buf0, buf1)
    sems = (sem0, sem1)

    def _start(g, b):
        row0 = pl.multiple_of(wid * _TROWS + g * _CROWS, _CROWS)
        pltpu.async_copy(neg_hbm.at[pl.ds(row0, _CROWS)], bufs[b], sems[b])

    def _wait(b):
        pltpu.make_async_copy(neg_hbm.at[pl.ds(0, _CROWS)], bufs[b],
                              sems[b]).wait()

    _start(0, 0)
    _start(1, 1)

    def _process(buf):
        @plsc.parallel_loop(0, _CROWS, 1, unroll=2)
        def _(r):
            for u in range(_RUNITS):
                v = buf[r, pl.ds(u * 16, 16)]
                b = jnp.maximum(jnp.minimum(v * inv, float(_NBINS - 1)), 0.0)
                idx = b.astype(jnp.int32) * 16 + lanes
                plsc.addupdate_scatter(stage, [idx], ones)
                plsc.addupdate_scatter(stage, [idx + _HWORDS], v)

    def cbody(g, _):
        for b in range(2):
            _wait(b)
            _process(bufs[b])

            @pl.when(g + b + 2 < _NCHUNKS)
            def _():
                _start(g + b + 2, b)

        return 0

    lax.fori_loop(0, _NCHUNKS // 2, lambda g, c: cbody(2 * g, c), 0)
    if _NCHUNKS % 2:
        _wait(0)
        _process(bufs[0])
    pltpu.sync_copy(stage, out_hbm.at[wid])


_hist_call = functools.partial(
    pl.kernel,
    out_type=jax.ShapeDtypeStruct((_TILES, 2 * _HWORDS), jnp.float32),
    mesh=plsc.VectorSubcoreMesh(core_axis_name="c", subcore_axis_name="s",
                                num_cores=2),
    compiler_params=pltpu.CompilerParams(needs_layout_passes=False),
    scratch_types=[
        pltpu.VMEM((2 * _HWORDS,), jnp.float32),
        pltpu.VMEM((_CROWS, _COLS), jnp.float32),
        pltpu.VMEM((_CROWS, _COLS), jnp.float32),
        pltpu.VMEM((8, 16), jnp.float32),
        pltpu.SemaphoreType.DMA,
        pltpu.SemaphoreType.DMA,
    ],
)(_hist_body)


def _params(*vals):
    cols = [jnp.broadcast_to(jnp.asarray(v, jnp.float32), (16,)) for v in vals]
    cols += [jnp.zeros(16, jnp.float32)] * (8 - len(cols))
    return jnp.stack(cols)


def _suffix(x):
    return jnp.concatenate([jnp.cumsum(x[::-1])[::-1], jnp.zeros(1, x.dtype)])


def kernel(prob_pred, prob_map, prob_mask, prob_weight):
    pred2 = prob_pred.reshape(_ROWS, _COLS)
    map2 = prob_map.reshape(_ROWS, _COLS)
    mask2 = prob_mask.reshape(_ROWS, _COLS)
    w2 = prob_weight.reshape(_ROWS, _COLS)

    neg2d, stats = _loss_call(pred2, map2, mask2, w2)
    pos_sum = stats[0, 0]
    pc = stats[0, 1].astype(jnp.int32)
    nc = stats[0, 2].astype(jnp.int32)
    k = jnp.minimum(nc, pc * 3)
    kf = k.astype(jnp.float32)

    inv_w = jnp.float32(_NBINS / _HI)
    out = _hist_call(neg2d, _params(inv_w))
    h = out.reshape(_TILES, 2, _NBINS, 16)
    cnt = h[:, 0].sum(axis=(0, 2))
    sm = h[:, 1].sum(axis=(0, 2))
    s = _suffix(cnt)
    ss = _suffix(sm)
    c = jnp.sum(s[:_NBINS] >= kf).astype(jnp.int32) - 1
    rem = kf - s[c + 1]
    avg = sm[c] / jnp.maximum(cnt[c], 1.0)
    topk = ss[c + 1] + rem * avg

    denom = (pc + k).astype(jnp.float32) + jnp.float32(1e-6)
    return (pos_sum + topk) / denom
